# baseline (speedup 1.0000x reference)
import functools

import jax
import jax.numpy as jnp
from jax import lax
from jax.experimental import pallas as pl
from jax.experimental.pallas import tpu as pltpu
from jax.experimental.pallas import tpu_sc as plsc

BATCH = 16384
GRID = 6 * 5
TOK = BATCH * GRID
EMB = 32
FB = 3
OUT_D = EMB + FB

NC = 2
NS = 16
NW = NC * NS
TPW = TOK // NW
CHUNK = 1024
IDX_W = 128
IDX_R = CHUNK // IDX_W
NCHUNK = TPW // CHUNK
NGROUP = CHUNK // 16


def _build():
    mesh = plsc.VectorSubcoreMesh(core_axis_name="c", subcore_axis_name="s")

    @functools.partial(
        pl.kernel,
        mesh=mesh,
        out_type=jax.ShapeDtypeStruct((TOK, OUT_D), jnp.float32),
        compiler_params=pltpu.CompilerParams(
            use_tc_tiling_on_sc=False, needs_layout_passes=False
        ),
        scratch_types=[
            pltpu.VMEM((IDX_R, IDX_W), jnp.int32),
            pltpu.VMEM((CHUNK, EMB), jnp.float32),
            pltpu.VMEM((CHUNK, OUT_D), jnp.float32),
            pltpu.VMEM((CHUNK * FB,), jnp.float32),
            pltpu.SemaphoreType.DMA,
        ],
    )
    def sc_kernel(letters_hbm, fb_hbm, table_hbm, out_hbm,
                  idx_v, rows_v, out_v, fb_v, sem):
        wid = lax.axis_index("s") * NC + lax.axis_index("c")
        wbase = wid * TPW

        lane = lax.iota(jnp.int32, 16)
        fb_rows = []
        fb_cols = []
        for p in range(FB):
            m = lane + (p * 16)
            q = lax.shift_right_logical(m * 21846, 16)
            fb_rows.append(q)
            fb_cols.append(m - q * FB + EMB)

        def chunk_body(i, carry):
            base = pl.multiple_of(wbase + i * CHUNK, CHUNK)
            row0 = pl.multiple_of(base // IDX_W, IDX_R)
            pltpu.sync_copy(letters_hbm.at[pl.ds(row0, IDX_R)], idx_v)
            for j in range(IDX_R):
                pltpu.async_copy(
                    table_hbm.at[idx_v.at[j]],
                    rows_v.at[pl.ds(j * IDX_W, IDX_W)],
                    sem,
                )
            pltpu.sync_copy(fb_hbm.at[pl.ds(base * FB, CHUNK * FB)], fb_v)
            for j in range(IDX_R):
                pltpu.make_async_copy(
                    table_hbm.at[idx_v.at[j]],
                    rows_v.at[pl.ds(j * IDX_W, IDX_W)],
                    sem,
                ).wait()

            pltpu.sync_copy(out_v, out_hbm.at[pl.ds(base, CHUNK)])
            return carry

        lax.fori_loop(0, NCHUNK, chunk_body, 0)

    return sc_kernel


_sc_kernel = _build()


@jax.jit
def kernel(letter_tensor, feedback_tensor, meta_tensor, letter_embed_table):
    letters = letter_tensor.reshape(TOK // IDX_W, IDX_W)
    fb = feedback_tensor.reshape(TOK * FB)
    out = _sc_kernel(letters, fb, letter_embed_table)
    return out.reshape(BATCH, 6, GRID // 6, OUT_D), meta_tensor


# speedup vs baseline: 4.5646x; 4.5646x over previous
import functools

import jax
import jax.numpy as jnp
from jax import lax
from jax.experimental import pallas as pl
from jax.experimental.pallas import tpu as pltpu
from jax.experimental.pallas import tpu_sc as plsc

G6 = 6
P5 = 5
NUNIT = G6 * P5
BATCH = 16384
EMB = 32
FB = 3
OUT_D = EMB + FB
ALPHA = 26

NC = 2
NS = 16
NW = NC * NS
SPAN = BATCH // NW
NGRP = SPAN // 16


def _build():
    mesh = plsc.VectorSubcoreMesh(core_axis_name="c", subcore_axis_name="s")

    @functools.partial(
        pl.kernel,
        mesh=mesh,
        out_type=jax.ShapeDtypeStruct((G6, P5, OUT_D, BATCH), jnp.float32),
        compiler_params=pltpu.CompilerParams(
            use_tc_tiling_on_sc=True, needs_layout_passes=False
        ),
        scratch_types=[
            pltpu.VMEM((ALPHA * EMB,), jnp.float32),
            pltpu.VMEM((G6, SPAN), jnp.int32),
            pltpu.VMEM((FB, SPAN), jnp.float32),
            pltpu.VMEM((OUT_D, SPAN), jnp.float32),
            pltpu.SemaphoreType.DMA,
        ],
    )
    def sc_kernel(lt_hbm, fb_hbm, table_hbm, out_hbm,
                  table_v, letters_v, fb_v, out_v, sem):
        wid = lax.axis_index("s") * NC + lax.axis_index("c")
        b0 = pl.multiple_of(wid * SPAN, SPAN)

        pltpu.sync_copy(table_hbm, table_v)

        def unit_body(u, carry):
            g = u // P5
            p = u - g * P5
            pltpu.sync_copy(lt_hbm.at[p, :, pl.ds(b0, SPAN)], letters_v)
            pltpu.sync_copy(fb_hbm.at[g, p, :, pl.ds(b0, SPAN)], fb_v)

            def group_body(c, carry2):
                off = c * 16
                lvec = letters_v[g, pl.ds(off, 16)]
                eidx = lvec * EMB
                for e in range(EMB):
                    vals = plsc.load_gather(table_v, [eidx + e])
                    out_v[e, pl.ds(off, 16)] = vals
                for d in range(FB):
                    out_v[EMB + d, pl.ds(off, 16)] = fb_v[d, pl.ds(off, 16)]
                return carry2

            lax.fori_loop(0, NGRP, group_body, 0)

            pltpu.sync_copy(out_v, out_hbm.at[g, p, :, pl.ds(b0, SPAN)])
            return carry

        lax.fori_loop(0, NUNIT, unit_body, 0)

    return sc_kernel


_sc_kernel = _build()


@jax.jit
def kernel(letter_tensor, feedback_tensor, meta_tensor, letter_embed_table):
    lt = jnp.transpose(letter_tensor, (2, 1, 0))
    fbt = jnp.transpose(feedback_tensor, (1, 2, 3, 0))
    tflat = letter_embed_table.reshape(ALPHA * EMB)
    out = _sc_kernel(lt, fbt, tflat)
    return jnp.transpose(out, (3, 0, 1, 2)), meta_tensor


# speedup vs baseline: 5.4147x; 1.1862x over previous
import functools

import jax
import jax.numpy as jnp
from jax import lax
from jax.experimental import pallas as pl
from jax.experimental.pallas import tpu as pltpu
from jax.experimental.pallas import tpu_sc as plsc

G6 = 6
P5 = 5
BATCH = 16384
EMB = 32
FB = 3
OUT_D = EMB + FB
ALPHA = 26

NC = 2
NS = 16
NW = NC * NS
SPAN = BATCH // NW
NGRP = SPAN // 16


def _build():
    mesh = plsc.VectorSubcoreMesh(core_axis_name="c", subcore_axis_name="s")

    @functools.partial(
        pl.kernel,
        mesh=mesh,
        out_type=jax.ShapeDtypeStruct((G6, P5, OUT_D, BATCH), jnp.float32),
        compiler_params=pltpu.CompilerParams(
            use_tc_tiling_on_sc=True, needs_layout_passes=False
        ),
        scratch_types=[
            pltpu.VMEM((ALPHA * EMB,), jnp.float32),
            pltpu.VMEM((G6, SPAN), jnp.int32),
            pltpu.VMEM((OUT_D, SPAN), jnp.float32),
            pltpu.VMEM((OUT_D, SPAN), jnp.float32),
            pltpu.SemaphoreType.DMA,
            pltpu.SemaphoreType.DMA,
            pltpu.SemaphoreType.DMA,
            pltpu.SemaphoreType.DMA,
        ],
    )
    def sc_kernel(lt_hbm, fb_hbm, table_hbm, out_hbm,
                  table_v, letters_v, out_v0, out_v1, sf0, sf1, so0, so1):
        wid = lax.axis_index("s") * NC + lax.axis_index("c")
        b0 = pl.multiple_of(wid * SPAN, SPAN)
        out_vs = (out_v0, out_v1)
        sfs = (sf0, sf1)
        sos = (so0, so1)

        pltpu.sync_copy(table_hbm, table_v)

        def p_body(p, carry):
            pltpu.sync_copy(lt_hbm.at[p, :, pl.ds(b0, SPAN)], letters_v)

            for g in range(G6):
                slot = g % 2
                ov = out_vs[slot]
                if g >= 2:
                    pltpu.make_async_copy(
                        ov, out_hbm.at[g - 2, p, :, pl.ds(b0, SPAN)], sos[slot]
                    ).wait()
                pltpu.async_copy(
                    fb_hbm.at[g, p, :, pl.ds(b0, SPAN)],
                    ov.at[pl.ds(EMB, FB)],
                    sfs[slot],
                )

                def group_body(c, carry2, _ov=ov):
                    off = c * 16
                    lvec = letters_v[g, pl.ds(off, 16)]
                    eidx = lvec * EMB
                    for e in range(EMB):
                        _ov[e, pl.ds(off, 16)] = plsc.load_gather(
                            table_v, [eidx + e]
                        )
                    return carry2

                lax.fori_loop(0, NGRP, group_body, 0)

                pltpu.make_async_copy(
                    fb_hbm.at[g, p, :, pl.ds(b0, SPAN)],
                    ov.at[pl.ds(EMB, FB)],
                    sfs[slot],
                ).wait()
                pltpu.async_copy(
                    ov, out_hbm.at[g, p, :, pl.ds(b0, SPAN)], sos[slot]
                )

            for g in (G6 - 2, G6 - 1):
                pltpu.make_async_copy(
                    out_vs[g % 2], out_hbm.at[g, p, :, pl.ds(b0, SPAN)],
                    sos[g % 2],
                ).wait()
            return carry

        lax.fori_loop(0, P5, p_body, 0)

    return sc_kernel


_sc_kernel = _build()


@jax.jit
def kernel(letter_tensor, feedback_tensor, meta_tensor, letter_embed_table):
    lt = jnp.transpose(letter_tensor, (2, 1, 0))
    fbt = jnp.transpose(feedback_tensor, (1, 2, 3, 0))
    tflat = letter_embed_table.reshape(ALPHA * EMB)
    out = _sc_kernel(lt, fbt, tflat)
    return jnp.transpose(out, (3, 0, 1, 2)), meta_tensor


# speedup vs baseline: 8.6541x; 1.5983x over previous
import functools

import jax
import jax.numpy as jnp
from jax import lax
from jax.experimental import pallas as pl
from jax.experimental.pallas import tpu as pltpu
from jax.experimental.pallas import tpu_sc as plsc

G6 = 6
P5 = 5
BATCH = 16384
EMB = 32
FB = 3
OUT_D = EMB + FB
ALPHA = 26

NC = 2
NS = 16
NW = NC * NS
SPAN = BATCH // NW
NGRP = SPAN // 16


def _build():
    mesh = plsc.VectorSubcoreMesh(core_axis_name="c", subcore_axis_name="s")

    @functools.partial(
        pl.kernel,
        mesh=mesh,
        out_type=jax.ShapeDtypeStruct((G6, P5, OUT_D, BATCH), jnp.float32),
        compiler_params=pltpu.CompilerParams(
            use_tc_tiling_on_sc=True, needs_layout_passes=False
        ),
        scratch_types=[
            pltpu.VMEM((ALPHA * EMB,), jnp.float32),
            pltpu.VMEM((G6, SPAN), jnp.int32),
            pltpu.VMEM((OUT_D, SPAN), jnp.float32),
            pltpu.VMEM((OUT_D, SPAN), jnp.float32),
            pltpu.SemaphoreType.DMA,
            pltpu.SemaphoreType.DMA,
            pltpu.SemaphoreType.DMA,
            pltpu.SemaphoreType.DMA,
        ],
    )
    def sc_kernel(lt_hbm, fb_hbm, table_hbm, out_hbm,
                  table_v, letters_v, out_v0, out_v1, sf0, sf1, so0, so1):
        wid = lax.axis_index("s") * NC + lax.axis_index("c")
        b0 = pl.multiple_of(wid * SPAN, SPAN)
        out_vs = (out_v0, out_v1)
        sfs = (sf0, sf1)
        sos = (so0, so1)

        pltpu.sync_copy(table_hbm, table_v)

        def p_body(p, carry):
            pltpu.sync_copy(lt_hbm.at[p, :, pl.ds(b0, SPAN)], letters_v)

            for g in range(G6):
                slot = g % 2
                ov = out_vs[slot]
                if g >= 2:
                    pltpu.make_async_copy(
                        ov, out_hbm.at[g - 2, p, :, pl.ds(b0, SPAN)], sos[slot]
                    ).wait()
                pltpu.async_copy(
                    fb_hbm.at[g, p, :, pl.ds(b0, SPAN)],
                    ov.at[pl.ds(EMB, FB)],
                    sfs[slot],
                )

                @plsc.parallel_loop(0, SPAN, 16, unroll=2)
                def group_body(off, _ov=ov, _g=g):
                    lvec = letters_v[_g, pl.ds(off, 16)]
                    eidx = lvec * EMB
                    for e in range(EMB):
                        _ov[e, pl.ds(off, 16)] = plsc.load_gather(
                            table_v, [eidx + e]
                        )

                pltpu.make_async_copy(
                    fb_hbm.at[g, p, :, pl.ds(b0, SPAN)],
                    ov.at[pl.ds(EMB, FB)],
                    sfs[slot],
                ).wait()
                pltpu.async_copy(
                    ov, out_hbm.at[g, p, :, pl.ds(b0, SPAN)], sos[slot]
                )

            for g in (G6 - 2, G6 - 1):
                pltpu.make_async_copy(
                    out_vs[g % 2], out_hbm.at[g, p, :, pl.ds(b0, SPAN)],
                    sos[g % 2],
                ).wait()
            return carry

        lax.fori_loop(0, P5, p_body, 0)

    return sc_kernel


_sc_kernel = _build()


@jax.jit
def kernel(letter_tensor, feedback_tensor, meta_tensor, letter_embed_table):
    lt = jnp.transpose(letter_tensor, (2, 1, 0))
    fbt = jnp.transpose(feedback_tensor, (1, 2, 3, 0))
    tflat = letter_embed_table.reshape(ALPHA * EMB)
    out = _sc_kernel(lt, fbt, tflat)
    return jnp.transpose(out, (3, 0, 1, 2)), meta_tensor
